# baseline (speedup 1.0000x reference)
import functools

import jax
import jax.numpy as jnp
import numpy as np
from jax.experimental import pallas as pl
import jax.experimental.pallas.tpu as pltpu


def _pass_a_body(x_ref, w1_ref, b1_ref, w2_ref, adj_ref,
                 x1_ref, s2_ref, y2p_ref, s12_ref, stage_ref, *, cw, blb):
    i = pl.program_id(0)
    bl = adj_ref.shape[0]
    d_hid = w1_ref.shape[1]

    @pl.when(i == 0)
    def _():
        s12_ref[:, :d_hid] = jnp.dot(x_ref[...], w1_ref[...],
                                     preferred_element_type=jnp.float32)
        s12_ref[:, d_hid:] = jnp.zeros_like(s12_ref[:, d_hid:])

    y = jnp.dot(adj_ref[...], s12_ref[...],
                preferred_element_type=jnp.float32)
    x1 = jnp.maximum(y[:, :d_hid] + b1_ref[...], 0.0)
    s2 = jnp.dot(x1, w2_ref[...], preferred_element_type=jnp.float32)
    x1_ref[...] = x1
    s2_ref[...] = s2
    y2p_ref[...] = y[:, d_hid:]

    stage_ref[pl.ds(i * bl, bl), :] = s2
    old_f = (blb * ((i * bl) // blb)) // cw
    new_f = (blb * (((i + 1) * bl) // blb)) // cw

    @pl.when(new_f > old_f)
    def _():
        s12_ref[pl.ds(old_f * cw, cw), d_hid:] = stage_ref[pl.ds(old_f * cw, cw), :]


def _pass_b_body(idx_ref, rhs_ref, adj_ref, y2t_ref, acc_ref,
                 *, n, cw, ncb):
    t = pl.program_id(0)
    jc = idx_ref[1, t]

    @pl.when(t == 0)
    def _():
        acc_ref[...] = jnp.zeros_like(acc_ref)

    if n % cw:
        @pl.when(jc == ncb - 1)
        def _():
            adj_ref[:, (n % cw):] = jnp.zeros_like(adj_ref[:, (n % cw):])

    acc_ref[...] += jnp.dot(adj_ref[...].astype(jnp.bfloat16), rhs_ref[...],
                            preferred_element_type=jnp.float32)

    @pl.when(jc == ncb - 1)
    def _():
        y2t_ref[...] = acc_ref[...]
        acc_ref[...] = jnp.zeros_like(acc_ref)


def _pass_c_body(y2p_ref, y2t_ref, b2_ref, out_ref):
    y = y2p_ref[...] + y2t_ref[...] + b2_ref[...]
    m = jnp.max(y, axis=1, keepdims=True)
    z = y - m
    out_ref[...] = z - jnp.log(jnp.sum(jnp.exp(z), axis=1, keepdims=True))


@functools.partial(jax.jit, static_argnames=("bl", "cw", "blb"))
def _gcn(x, adj, W1, b1, W2, b2, bl=200, cw=1024, blb=1000):
    n, d_in = x.shape
    d_hid = W1.shape[1]
    d_out = W2.shape[1]
    nrb = n // bl
    ncb = -(-n // cw)
    nrbb = n // blb
    assert blb % bl == 0 and blb <= cw

    full = lambda s: pl.BlockSpec(s, lambda *_: (0,) * len(s))

    x1, s2, y2p = pl.pallas_call(
        functools.partial(_pass_a_body, cw=cw, blb=blb),
        grid=(nrb,),
        in_specs=[full((n, d_in)), full((d_in, d_hid)), full((1, d_hid)),
                  full((d_hid, d_out)),
                  pl.BlockSpec((bl, n), lambda i: (i, 0))],
        out_specs=[pl.BlockSpec((bl, d_hid), lambda i: (i, 0)),
                   pl.BlockSpec((bl, d_out), lambda i: (i, 0)),
                   pl.BlockSpec((bl, d_out), lambda i: (i, 0))],
        out_shape=[jax.ShapeDtypeStruct((n, d_hid), jnp.float32),
                   jax.ShapeDtypeStruct((n, d_out), jnp.float32),
                   jax.ShapeDtypeStruct((n, d_out), jnp.float32)],
        scratch_shapes=[pltpu.VMEM((n, d_hid + d_out), jnp.float32),
                        pltpu.VMEM((n, d_out), jnp.float32)],
    )(x, W1, b1.reshape(1, d_hid), W2, adj)

    steps = [(i, jc) for i in range(nrbb)
             for jc in range((i * blb) // cw, ncb)]
    idx = jnp.asarray(np.array(steps, dtype=np.int32).T)

    s2p = jnp.zeros((ncb * cw, d_out), jnp.bfloat16).at[:n].set(
        s2.astype(jnp.bfloat16))

    grid_spec = pltpu.PrefetchScalarGridSpec(
        num_scalar_prefetch=1,
        grid=(len(steps),),
        in_specs=[pl.BlockSpec((cw, d_out), lambda t, ix: (ix[1, t], 0)),
                  pl.BlockSpec((blb, cw), lambda t, ix: (ix[0, t], ix[1, t]))],
        out_specs=pl.BlockSpec((blb, d_out), lambda t, ix: (ix[0, t], 0)),
        scratch_shapes=[pltpu.VMEM((blb, d_out), jnp.float32)],
    )

    y2t = pl.pallas_call(
        functools.partial(_pass_b_body, n=n, cw=cw, ncb=ncb),
        grid_spec=grid_spec,
        out_shape=jax.ShapeDtypeStruct((n, d_out), jnp.float32),
    )(idx, s2p, adj)

    out = pl.pallas_call(
        _pass_c_body,
        in_specs=[full((n, d_out)), full((n, d_out)), full((1, d_out))],
        out_specs=full((n, d_out)),
        out_shape=jax.ShapeDtypeStruct((n, d_out), jnp.float32),
    )(y2p, y2t, b2.reshape(1, d_out))

    return out, x1


def kernel(x, adj, W1, b1, W2, b2):
    return _gcn(x, adj, W1, b1, W2, b2)


# speedup vs baseline: 1.0746x; 1.0746x over previous
import functools

import jax
import jax.numpy as jnp
import numpy as np
from jax.experimental import pallas as pl
import jax.experimental.pallas.tpu as pltpu


def _pass_a_body(x_ref, w1_ref, b1_ref, w2_ref, adj_ref,
                 x1_ref, s2_ref, y2p_ref, s12_ref, stage_ref, *, cw, blb):
    i = pl.program_id(0)
    bl = adj_ref.shape[0]
    d_hid = w1_ref.shape[1]

    @pl.when(i == 0)
    def _():
        s12_ref[:, :d_hid] = jnp.dot(x_ref[...], w1_ref[...],
                                     preferred_element_type=jnp.float32)
        s12_ref[:, d_hid:] = jnp.zeros_like(s12_ref[:, d_hid:])

    y = jnp.dot(adj_ref[...], s12_ref[...],
                preferred_element_type=jnp.float32)
    x1 = jnp.maximum(y[:, :d_hid] + b1_ref[...], 0.0)
    s2 = jnp.dot(x1, w2_ref[...], preferred_element_type=jnp.float32)
    x1_ref[...] = x1
    s2_ref[...] = s2
    y2p_ref[...] = y[:, d_hid:]

    stage_ref[pl.ds(i * bl, bl), :] = s2
    old_f = (blb * ((i * bl) // blb)) // cw
    new_f = (blb * (((i + 1) * bl) // blb)) // cw
    for k in range(-(-blb // cw)):
        @pl.when(new_f > old_f + k)
        def _(k=k):
            s12_ref[pl.ds((old_f + k) * cw, cw), d_hid:] = \
                stage_ref[pl.ds((old_f + k) * cw, cw), :]


def _pass_b_body(idx_ref, rhs_ref, adj_ref, y2t_ref, acc_ref,
                 *, n, cw, ncb):
    t = pl.program_id(0)
    jc = idx_ref[1, t]

    @pl.when(t == 0)
    def _():
        acc_ref[...] = jnp.zeros_like(acc_ref)

    if n % cw:
        @pl.when(jc == ncb - 1)
        def _():
            adj_ref[:, (n % cw):] = jnp.zeros_like(adj_ref[:, (n % cw):])

    acc_ref[...] += jnp.dot(adj_ref[...].astype(jnp.bfloat16), rhs_ref[...],
                            preferred_element_type=jnp.float32)

    @pl.when(jc == ncb - 1)
    def _():
        y2t_ref[...] = acc_ref[...]
        acc_ref[...] = jnp.zeros_like(acc_ref)


def _pass_c_body(y2p_ref, y2t_ref, b2_ref, out_ref):
    y = y2p_ref[...] + y2t_ref[...] + b2_ref[...]
    m = jnp.max(y, axis=1, keepdims=True)
    z = y - m
    out_ref[...] = z - jnp.log(jnp.sum(jnp.exp(z), axis=1, keepdims=True))


@functools.partial(jax.jit, static_argnames=("bl", "cw", "blb"))
def _gcn(x, adj, W1, b1, W2, b2, bl=400, cw=1024, blb=2000):
    n, d_in = x.shape
    d_hid = W1.shape[1]
    d_out = W2.shape[1]
    nrb = n // bl
    ncb = -(-n // cw)
    nrbb = n // blb
    assert blb % bl == 0

    full = lambda s: pl.BlockSpec(s, lambda *_: (0,) * len(s))

    x1, s2, y2p = pl.pallas_call(
        functools.partial(_pass_a_body, cw=cw, blb=blb),
        grid=(nrb,),
        in_specs=[full((n, d_in)), full((d_in, d_hid)), full((1, d_hid)),
                  full((d_hid, d_out)),
                  pl.BlockSpec((bl, n), lambda i: (i, 0))],
        out_specs=[pl.BlockSpec((bl, d_hid), lambda i: (i, 0)),
                   pl.BlockSpec((bl, d_out), lambda i: (i, 0)),
                   pl.BlockSpec((bl, d_out), lambda i: (i, 0))],
        out_shape=[jax.ShapeDtypeStruct((n, d_hid), jnp.float32),
                   jax.ShapeDtypeStruct((n, d_out), jnp.float32),
                   jax.ShapeDtypeStruct((n, d_out), jnp.float32)],
        scratch_shapes=[pltpu.VMEM((n, d_hid + d_out), jnp.float32),
                        pltpu.VMEM((n, d_out), jnp.float32)],
        compiler_params=pltpu.CompilerParams(vmem_limit_bytes=127 * 1024 * 1024),
    )(x, W1, b1.reshape(1, d_hid), W2, adj)

    steps = [(i, jc) for i in range(nrbb)
             for jc in range((i * blb) // cw, ncb)]
    idx = jnp.asarray(np.array(steps, dtype=np.int32).T)

    s2p = jnp.zeros((ncb * cw, d_out), jnp.bfloat16).at[:n].set(
        s2.astype(jnp.bfloat16))

    grid_spec = pltpu.PrefetchScalarGridSpec(
        num_scalar_prefetch=1,
        grid=(len(steps),),
        in_specs=[pl.BlockSpec((cw, d_out), lambda t, ix: (ix[1, t], 0)),
                  pl.BlockSpec((blb, cw), lambda t, ix: (ix[0, t], ix[1, t]))],
        out_specs=pl.BlockSpec((blb, d_out), lambda t, ix: (ix[0, t], 0)),
        scratch_shapes=[pltpu.VMEM((blb, d_out), jnp.float32)],
    )

    y2t = pl.pallas_call(
        functools.partial(_pass_b_body, n=n, cw=cw, ncb=ncb),
        grid_spec=grid_spec,
        out_shape=jax.ShapeDtypeStruct((n, d_out), jnp.float32),
    )(idx, s2p, adj)

    out = pl.pallas_call(
        _pass_c_body,
        in_specs=[full((n, d_out)), full((n, d_out)), full((1, d_out))],
        out_specs=full((n, d_out)),
        out_shape=jax.ShapeDtypeStruct((n, d_out), jnp.float32),
    )(y2p, y2t, b2.reshape(1, d_out))

    return out, x1


def kernel(x, adj, W1, b1, W2, b2):
    return _gcn(x, adj, W1, b1, W2, b2)


# speedup vs baseline: 1.1016x; 1.0251x over previous
import functools

import jax
import jax.numpy as jnp
import numpy as np
from jax.experimental import pallas as pl
import jax.experimental.pallas.tpu as pltpu


def _pass_a_body(x_ref, w1_ref, b1_ref, w2_ref, adj_ref,
                 x1_ref, s2_ref, y2p_ref, s12_ref, stage_ref, *, cw, blb):
    i = pl.program_id(0)
    bl = adj_ref.shape[0]
    d_hid = w1_ref.shape[1]

    @pl.when(i == 0)
    def _():
        s12_ref[:, :d_hid] = jnp.dot(x_ref[...], w1_ref[...],
                                     preferred_element_type=jnp.float32)
        s12_ref[:, d_hid:] = jnp.zeros_like(s12_ref[:, d_hid:])

    y = jnp.dot(adj_ref[...], s12_ref[...],
                preferred_element_type=jnp.float32)
    x1 = jnp.maximum(y[:, :d_hid] + b1_ref[...], 0.0)
    s2 = jnp.dot(x1, w2_ref[...], preferred_element_type=jnp.float32)
    x1_ref[...] = x1
    s2_ref[...] = s2
    y2p_ref[...] = y[:, d_hid:]

    stage_ref[pl.ds(i * bl, bl), :] = s2
    old_f = (blb * ((i * bl) // blb)) // cw
    new_f = (blb * (((i + 1) * bl) // blb)) // cw
    for k in range(-(-blb // cw)):
        @pl.when(new_f > old_f + k)
        def _(k=k):
            s12_ref[pl.ds((old_f + k) * cw, cw), d_hid:] = \
                stage_ref[pl.ds((old_f + k) * cw, cw), :]


def _pass_b_body(idx_ref, rhs_ref, adj_ref, y2t_ref, acc_ref,
                 *, n, cw, ncb):
    t = pl.program_id(0)
    jc = idx_ref[1, t]

    @pl.when(t == 0)
    def _():
        acc_ref[...] = jnp.zeros_like(acc_ref)

    if n % cw:
        @pl.when(jc == ncb - 1)
        def _():
            adj_ref[:, (n % cw):] = jnp.zeros_like(adj_ref[:, (n % cw):])

    acc_ref[...] += jnp.dot(adj_ref[...].astype(jnp.bfloat16), rhs_ref[...],
                            preferred_element_type=jnp.float32)

    @pl.when(jc == ncb - 1)
    def _():
        y2t_ref[...] = acc_ref[...]
        acc_ref[...] = jnp.zeros_like(acc_ref)


def _pass_c_body(y2p_ref, y2t_ref, b2_ref, out_ref):
    y = y2p_ref[...] + y2t_ref[...] + b2_ref[...]
    m = jnp.max(y, axis=1, keepdims=True)
    z = y - m
    out_ref[...] = z - jnp.log(jnp.sum(jnp.exp(z), axis=1, keepdims=True))


@functools.partial(jax.jit, static_argnames=("bl", "cw", "blb"))
def _gcn(x, adj, W1, b1, W2, b2, bl=400, cw=1280, blb=2000):
    n, d_in = x.shape
    d_hid = W1.shape[1]
    d_out = W2.shape[1]
    nrb = n // bl
    ncb = -(-n // cw)
    nrbb = n // blb
    assert blb % bl == 0

    full = lambda s: pl.BlockSpec(s, lambda *_: (0,) * len(s))

    x1, s2, y2p = pl.pallas_call(
        functools.partial(_pass_a_body, cw=cw, blb=blb),
        grid=(nrb,),
        in_specs=[full((n, d_in)), full((d_in, d_hid)), full((1, d_hid)),
                  full((d_hid, d_out)),
                  pl.BlockSpec((bl, n), lambda i: (i, 0))],
        out_specs=[pl.BlockSpec((bl, d_hid), lambda i: (i, 0)),
                   pl.BlockSpec((bl, d_out), lambda i: (i, 0)),
                   pl.BlockSpec((bl, d_out), lambda i: (i, 0))],
        out_shape=[jax.ShapeDtypeStruct((n, d_hid), jnp.float32),
                   jax.ShapeDtypeStruct((n, d_out), jnp.float32),
                   jax.ShapeDtypeStruct((n, d_out), jnp.float32)],
        scratch_shapes=[pltpu.VMEM((n, d_hid + d_out), jnp.float32),
                        pltpu.VMEM((n, d_out), jnp.float32)],
        compiler_params=pltpu.CompilerParams(vmem_limit_bytes=127 * 1024 * 1024),
    )(x, W1, b1.reshape(1, d_hid), W2, adj)

    steps = [(i, jc) for i in range(nrbb)
             for jc in range((i * blb) // cw, ncb)]
    idx = jnp.asarray(np.array(steps, dtype=np.int32).T)

    s2p = jnp.zeros((ncb * cw, d_out), jnp.bfloat16).at[:n].set(
        s2.astype(jnp.bfloat16))

    grid_spec = pltpu.PrefetchScalarGridSpec(
        num_scalar_prefetch=1,
        grid=(len(steps),),
        in_specs=[pl.BlockSpec((cw, d_out), lambda t, ix: (ix[1, t], 0)),
                  pl.BlockSpec((blb, cw), lambda t, ix: (ix[0, t], ix[1, t]))],
        out_specs=pl.BlockSpec((blb, d_out), lambda t, ix: (ix[0, t], 0)),
        scratch_shapes=[pltpu.VMEM((blb, d_out), jnp.float32)],
    )

    y2t = pl.pallas_call(
        functools.partial(_pass_b_body, n=n, cw=cw, ncb=ncb),
        grid_spec=grid_spec,
        out_shape=jax.ShapeDtypeStruct((n, d_out), jnp.float32),
    )(idx, s2p, adj)

    out = pl.pallas_call(
        _pass_c_body,
        in_specs=[full((n, d_out)), full((n, d_out)), full((1, d_out))],
        out_specs=full((n, d_out)),
        out_shape=jax.ShapeDtypeStruct((n, d_out), jnp.float32),
    )(y2p, y2t, b2.reshape(1, d_out))

    return out, x1


def kernel(x, adj, W1, b1, W2, b2):
    return _gcn(x, adj, W1, b1, W2, b2)


# speedup vs baseline: 1.1248x; 1.0211x over previous
import functools

import jax
import jax.numpy as jnp
import numpy as np
from jax.experimental import pallas as pl
import jax.experimental.pallas.tpu as pltpu


def _pass_a_body(x_ref, w1_ref, b1_ref, w2_ref, adj_ref,
                 x1_ref, s2_ref, y2p_ref, s12_ref, stage_ref, *, cw, blb):
    i = pl.program_id(0)
    bl = adj_ref.shape[0]
    d_hid = w1_ref.shape[1]

    @pl.when(i == 0)
    def _():
        s12_ref[:, :d_hid] = jnp.dot(x_ref[...], w1_ref[...],
                                     preferred_element_type=jnp.float32)
        s12_ref[:, d_hid:] = jnp.zeros_like(s12_ref[:, d_hid:])

    y = jnp.dot(adj_ref[...], s12_ref[...],
                preferred_element_type=jnp.float32)
    x1 = jnp.maximum(y[:, :d_hid] + b1_ref[...], 0.0)
    s2 = jnp.dot(x1, w2_ref[...], preferred_element_type=jnp.float32)
    x1_ref[...] = x1
    s2_ref[...] = s2
    y2p_ref[...] = y[:, d_hid:]

    stage_ref[pl.ds(i * bl, bl), :] = s2
    old_f = (blb * ((i * bl) // blb)) // cw
    new_f = (blb * (((i + 1) * bl) // blb)) // cw
    for k in range(-(-blb // cw)):
        @pl.when(new_f > old_f + k)
        def _(k=k):
            s12_ref[pl.ds((old_f + k) * cw, cw), d_hid:] = \
                stage_ref[pl.ds((old_f + k) * cw, cw), :]


def _pass_b_body(idx_ref, rhs_ref, adj_ref, y2t_ref, acc_ref,
                 *, n, cw, ncb):
    t = pl.program_id(0)
    jc = idx_ref[1, t]

    @pl.when(t == 0)
    def _():
        acc_ref[...] = jnp.zeros_like(acc_ref)

    if n % cw:
        @pl.when(jc == ncb - 1)
        def _():
            adj_ref[:, (n % cw):] = jnp.zeros_like(adj_ref[:, (n % cw):])

    acc_ref[...] += jnp.dot(adj_ref[...].astype(jnp.bfloat16), rhs_ref[...],
                            preferred_element_type=jnp.float32)

    @pl.when(jc == ncb - 1)
    def _():
        y2t_ref[...] = acc_ref[...]
        acc_ref[...] = jnp.zeros_like(acc_ref)


def _pass_c_body(y2p_ref, y2t_ref, b2_ref, out_ref):
    y = y2p_ref[...] + y2t_ref[...] + b2_ref[...]
    m = jnp.max(y, axis=1, keepdims=True)
    z = y - m
    out_ref[...] = z - jnp.log(jnp.sum(jnp.exp(z), axis=1, keepdims=True))


@functools.partial(jax.jit, static_argnames=("bl", "cw", "blb"))
def _gcn(x, adj, W1, b1, W2, b2, bl=400, cw=1280, blb=2000):
    n, d_in = x.shape
    d_hid = W1.shape[1]
    d_out = W2.shape[1]
    nrb = n // bl
    ncb = -(-n // cw)
    nrbb = n // blb
    assert blb % bl == 0

    full = lambda s: pl.BlockSpec(s, lambda *_: (0,) * len(s))

    x1, s2, y2p = pl.pallas_call(
        functools.partial(_pass_a_body, cw=cw, blb=blb),
        grid=(nrb,),
        in_specs=[full((n, d_in)), full((d_in, d_hid)), full((1, d_hid)),
                  full((d_hid, d_out)),
                  pl.BlockSpec((bl, n), lambda i: (i, 0))],
        out_specs=[pl.BlockSpec((bl, d_hid), lambda i: (i, 0)),
                   pl.BlockSpec((bl, d_out), lambda i: (i, 0)),
                   pl.BlockSpec((bl, d_out), lambda i: (i, 0))],
        out_shape=[jax.ShapeDtypeStruct((n, d_hid), jnp.float32),
                   jax.ShapeDtypeStruct((n, d_out), jnp.float32),
                   jax.ShapeDtypeStruct((n, d_out), jnp.float32)],
        scratch_shapes=[pltpu.VMEM((n, d_hid + d_out), jnp.float32),
                        pltpu.VMEM((n, d_out), jnp.float32)],
    )(x, W1, b1.reshape(1, d_hid), W2, adj)

    steps = [(i, jc) for i in range(nrbb)
             for jc in range((i * blb) // cw, ncb)]
    idx = jnp.asarray(np.array(steps, dtype=np.int32).T)

    s2p = jnp.zeros((ncb * cw, d_out), jnp.bfloat16).at[:n].set(
        s2.astype(jnp.bfloat16))

    grid_spec = pltpu.PrefetchScalarGridSpec(
        num_scalar_prefetch=1,
        grid=(len(steps),),
        in_specs=[pl.BlockSpec((cw, d_out), lambda t, ix: (ix[1, t], 0)),
                  pl.BlockSpec((blb, cw), lambda t, ix: (ix[0, t], ix[1, t]))],
        out_specs=pl.BlockSpec((blb, d_out), lambda t, ix: (ix[0, t], 0)),
        scratch_shapes=[pltpu.VMEM((blb, d_out), jnp.float32)],
    )

    y2t = pl.pallas_call(
        functools.partial(_pass_b_body, n=n, cw=cw, ncb=ncb),
        grid_spec=grid_spec,
        out_shape=jax.ShapeDtypeStruct((n, d_out), jnp.float32),
    )(idx, s2p, adj)

    out = pl.pallas_call(
        _pass_c_body,
        in_specs=[full((n, d_out)), full((n, d_out)), full((1, d_out))],
        out_specs=full((n, d_out)),
        out_shape=jax.ShapeDtypeStruct((n, d_out), jnp.float32),
    )(y2p, y2t, b2.reshape(1, d_out))

    return out, x1


def kernel(x, adj, W1, b1, W2, b2):
    return _gcn(x, adj, W1, b1, W2, b2)
